# Initial kernel scaffold; baseline (speedup 1.0000x reference)
import jax
import jax.numpy as jnp
from jax.experimental import pallas as pl


def kernel(x, edge_index, edge_weight, l1_W0x, l1_W1x, l1_bx, l1_W0h, l1_W1h, l1_bh, l1_wc, l1_bg, l2_W0x, l2_W1x, l2_bx, l2_W0h, l2_W1h, l2_bh, l2_wc, l2_bg, lin_W, lin_b):
    raise NotImplementedError("write your pallas kernel here")


# speedup vs baseline: 8.8358x; 8.8358x over previous
import functools

import jax
import jax.numpy as jnp
from jax import lax
from jax.experimental import pallas as pl
from jax.experimental.pallas import tpu as pltpu
from jax.experimental.pallas import tpu_sc as plsc

N = 10000
E = 320000
D_IN = 128
H1 = 50
H2 = 20
H2PAD = 128

NC = 2
NS = 16
NW = NC * NS
EW = E // NW
K = 80
NCH = EW // K
TB = 624
TROWS = 640
ZR = 128

def _sc_mesh():
  return plsc.VectorSubcoreMesh(core_axis_name="c", subcore_axis_name="s")


def _make_lap_sc(D):
  def body(y_hbm, w_hbm, row_hbm, col_hbm, out_hbm,
           ridx_v, cidx_v, w_v, rows_v, zero_v, acc_sh, gsem):
    cid = lax.axis_index("c")
    sid = lax.axis_index("s")
    wid = sid * NC + cid

    def zslab(i, _):
      for d in range(D // 16):
        zero_v[i, pl.ds(d * 16, 16)] = jnp.zeros((16,), jnp.float32)
      return 0
    lax.fori_loop(0, ZR, zslab, 0)

    def zacc(i, _):
      pltpu.sync_copy(zero_v, acc_sh.at[pl.ds(sid * TB + i * ZR, ZR)])
      return 0
    lax.fori_loop(0, TROWS // ZR, zacc, 0)
    plsc.subcore_barrier()

    base0 = wid * EW

    def chunk(j, _):
      base = base0 + j * K
      pltpu.sync_copy(col_hbm.at[pl.ds(base, K)], cidx_v)
      pltpu.sync_copy(w_hbm.at[pl.ds(base, K)], w_v)
      pltpu.sync_copy(row_hbm.at[pl.ds(base, K)], ridx_v)
      pltpu.async_copy(y_hbm.at[ridx_v], rows_v, gsem).wait()

      def scale(m, _):
        w16 = w_v[pl.ds(m * 16, 16)]
        for l in range(16):
          k = m * 16 + l
          wb = jnp.full((16,), w16[l], jnp.float32)
          for d in range(D // 16):
            rows_v[k, pl.ds(d * 16, 16)] = rows_v[k, pl.ds(d * 16, 16)] * wb
        return 0
      lax.fori_loop(0, K // 16, scale, 0)

      pltpu.sync_copy(rows_v, acc_sh.at[cidx_v], add=True)
      return 0
    lax.fori_loop(0, NCH, chunk, 0)

    plsc.subcore_barrier()
    pltpu.sync_copy(acc_sh.at[pl.ds(sid * TB, TROWS)],
                    out_hbm.at[cid, pl.ds(sid * TB, TROWS)])

  return functools.partial(
      pl.kernel,
      mesh=_sc_mesh(),
      out_type=jax.ShapeDtypeStruct((NC, N, D), jnp.float32),
      scratch_types=[
          pltpu.VMEM((K,), jnp.int32),
          pltpu.VMEM((K,), jnp.int32),
          pltpu.VMEM((K,), jnp.float32),
          pltpu.VMEM((K, D), jnp.float32),
          pltpu.VMEM((ZR, D), jnp.float32),
          pltpu.VMEM_SHARED((N, D), jnp.float32),
          pltpu.SemaphoreType.DMA,
      ],
  )(body)


def _make_deg_sc():
  D = 128

  def body(w_hbm, row_hbm, out_hbm, ridx_v, w_v, rows_v, zero_v, acc_sh):
    cid = lax.axis_index("c")
    sid = lax.axis_index("s")
    wid = sid * NC + cid

    def zslab(i, _):
      for d in range(D // 16):
        zero_v[i, pl.ds(d * 16, 16)] = jnp.zeros((16,), jnp.float32)
      return 0
    lax.fori_loop(0, ZR, zslab, 0)

    def zacc(i, _):
      pltpu.sync_copy(zero_v, acc_sh.at[pl.ds(sid * TB + i * ZR, ZR)])
      return 0
    lax.fori_loop(0, TROWS // ZR, zacc, 0)
    plsc.subcore_barrier()

    base0 = wid * EW

    def chunk(j, _):
      base = base0 + j * K
      pltpu.sync_copy(row_hbm.at[pl.ds(base, K)], ridx_v)
      pltpu.sync_copy(w_hbm.at[pl.ds(base, K)], w_v)

      def fill(m, _):
        w16 = w_v[pl.ds(m * 16, 16)]
        for l in range(16):
          wb = jnp.full((16,), w16[l], jnp.float32)
          for d in range(D // 16):
            rows_v[m * 16 + l, pl.ds(d * 16, 16)] = wb
        return 0
      lax.fori_loop(0, K // 16, fill, 0)

      pltpu.sync_copy(rows_v, acc_sh.at[ridx_v], add=True)
      return 0
    lax.fori_loop(0, NCH, chunk, 0)

    plsc.subcore_barrier()
    pltpu.sync_copy(acc_sh.at[pl.ds(sid * TB, TROWS)],
                    out_hbm.at[cid, pl.ds(sid * TB, TROWS)])

  return functools.partial(
      pl.kernel,
      mesh=_sc_mesh(),
      out_type=jax.ShapeDtypeStruct((NC, N, D), jnp.float32),
      scratch_types=[
          pltpu.VMEM((K,), jnp.int32),
          pltpu.VMEM((K,), jnp.float32),
          pltpu.VMEM((K, D), jnp.float32),
          pltpu.VMEM((ZR, D), jnp.float32),
          pltpu.VMEM_SHARED((N, D), jnp.float32),
      ],
  )(body)


BN = 2000


def _tc_pre(degp_ref, x_ref, dis_ref, y1_ref):
  d16 = degp_ref[0, :, :16] + degp_ref[1, :, :16]
  dis = jnp.where(d16 > 0, lax.rsqrt(jnp.maximum(d16, 1e-12)),
                  jnp.zeros_like(d16))
  dis_ref[...] = dis
  y1_ref[...] = dis[:, :1] * x_ref[...]


def _tc_cell1(x_ref, aggp_ref, dis_ref, w0_ref, w1_ref, b_ref, wc2_ref,
              h1_ref, y2_ref):
  dis = dis_ref[:, :1]
  lx = (-dis) * (aggp_ref[0] + aggp_ref[1])
  g = (jnp.dot(x_ref[...], w0_ref[...], preferred_element_type=jnp.float32)
       + jnp.dot(lx, w1_ref[...], preferred_element_type=jnp.float32)
       + b_ref[...])
  i = jax.nn.sigmoid(g[:, :H1])
  t = jnp.tanh(g[:, H1:2 * H1])
  c = i * t
  o = jax.nn.sigmoid(g[:, 2 * H1:3 * H1] + wc2_ref[...] * c)
  h1 = jax.nn.relu(o * jnp.tanh(c))
  h1p = jnp.concatenate(
      [h1, jnp.zeros((h1.shape[0], H2PAD - H1), h1.dtype)], axis=1)
  h1_ref[...] = h1p
  y2_ref[...] = dis * h1p


def _tc_cell2(h1_ref, aggp_ref, dis_ref, w0_ref, w1_ref, b_ref, wc2_ref,
              lw_ref, lb_ref, out_ref):
  dis = dis_ref[:, :1]
  lx = (-dis) * (aggp_ref[0] + aggp_ref[1])
  g = (jnp.dot(h1_ref[...], w0_ref[...], preferred_element_type=jnp.float32)
       + jnp.dot(lx, w1_ref[...], preferred_element_type=jnp.float32)
       + b_ref[...])
  i = jax.nn.sigmoid(g[:, :H2])
  t = jnp.tanh(g[:, H2:2 * H2])
  c = i * t
  o = jax.nn.sigmoid(g[:, 2 * H2:3 * H2] + wc2_ref[...] * c)
  h2 = jax.nn.relu(o * jnp.tanh(c))
  out_ref[...] = (jnp.dot(h2, lw_ref[...], preferred_element_type=jnp.float32)
                  + lb_ref[...])


def _row_spec(w):
  return pl.BlockSpec((BN, w), lambda i: (i, 0))


def _full_spec(shape):
  return pl.BlockSpec(shape, lambda i: tuple(0 for _ in shape))


def _aggp_spec(w):
  return pl.BlockSpec((NC, BN, w), lambda i: (0, i, 0))


_GRID = (N // BN,)


def kernel(x, edge_index, edge_weight, l1_W0x, l1_W1x, l1_bx, l1_W0h, l1_W1h,
           l1_bh, l1_wc, l1_bg, l2_W0x, l2_W1x, l2_bx, l2_W0h, l2_W1h, l2_bh,
           l2_wc, l2_bg, lin_W, lin_b):
  row = edge_index[0]
  col = edge_index[1]

  gsel = jnp.array([0, 2, 3], jnp.int32)
  w0c1 = jnp.concatenate([l1_W0x[g] for g in (0, 2, 3)], axis=1)
  w1c1 = jnp.concatenate([l1_W1x[g] for g in (0, 2, 3)], axis=1)
  b1 = (l1_bx + l1_bh + l1_bg)[gsel].reshape(1, 3 * H1)
  wc21 = l1_wc[2].reshape(1, H1)

  zpad = jnp.zeros((H2PAD - H1, 3 * H2), jnp.float32)
  w0c2 = jnp.concatenate(
      [jnp.concatenate([l2_W0x[g] for g in (0, 2, 3)], axis=1), zpad], axis=0)
  w1c2 = jnp.concatenate(
      [jnp.concatenate([l2_W1x[g] for g in (0, 2, 3)], axis=1), zpad], axis=0)
  b2 = (l2_bx + l2_bh + l2_bg)[gsel].reshape(1, 3 * H2)
  wc22 = l2_wc[2].reshape(1, H2)
  linb = lin_b.reshape(1, 1)

  degp = _make_deg_sc()(edge_weight, row)

  dis16, y1 = pl.pallas_call(
      _tc_pre,
      grid=_GRID,
      in_specs=[_aggp_spec(D_IN), _row_spec(D_IN)],
      out_specs=[_row_spec(16), _row_spec(D_IN)],
      out_shape=[jax.ShapeDtypeStruct((N, 16), jnp.float32),
                 jax.ShapeDtypeStruct((N, D_IN), jnp.float32)],
  )(degp, x)

  agg1 = _make_lap_sc(D_IN)(y1, edge_weight, row, col)

  h1p, y2 = pl.pallas_call(
      _tc_cell1,
      grid=_GRID,
      in_specs=[_row_spec(D_IN), _aggp_spec(D_IN), _row_spec(16),
                _full_spec((D_IN, 3 * H1)), _full_spec((D_IN, 3 * H1)),
                _full_spec((1, 3 * H1)), _full_spec((1, H1))],
      out_specs=[_row_spec(H2PAD), _row_spec(H2PAD)],
      out_shape=[jax.ShapeDtypeStruct((N, H2PAD), jnp.float32),
                 jax.ShapeDtypeStruct((N, H2PAD), jnp.float32)],
  )(x, agg1, dis16, w0c1, w1c1, b1, wc21)

  agg2 = _make_lap_sc(H2PAD)(y2, edge_weight, row, col)

  out = pl.pallas_call(
      _tc_cell2,
      grid=_GRID,
      in_specs=[_row_spec(H2PAD), _aggp_spec(H2PAD), _row_spec(16),
                _full_spec((H2PAD, 3 * H2)), _full_spec((H2PAD, 3 * H2)),
                _full_spec((1, 3 * H2)), _full_spec((1, H2)),
                _full_spec((H2, 1)), _full_spec((1, 1))],
      out_specs=_row_spec(1),
      out_shape=jax.ShapeDtypeStruct((N, 1), jnp.float32),
  )(h1p, agg2, dis16, w0c2, w1c2, b2, wc22, lin_W, linb)

  return out


# speedup vs baseline: 8.9630x; 1.0144x over previous
import functools

import jax
import jax.numpy as jnp
from jax import lax
from jax.experimental import pallas as pl
from jax.experimental.pallas import tpu as pltpu
from jax.experimental.pallas import tpu_sc as plsc

N = 10000
E = 320000
D_IN = 128
H1 = 50
H2 = 20
H2PAD = 128

NC = 2
NS = 16
NW = NC * NS
K = 128
NCH = 80
IB = 16
NSTG = NCH // IB
EPAD = NW * NCH * K
TB = 624
TROWS = 640

def _sc_mesh():
  return plsc.VectorSubcoreMesh(core_axis_name="c", subcore_axis_name="s")


def _scale_chunk(rows_v, w_row, D):
  def scale(m, _):
    w16 = w_row[pl.ds(m * 16, 16)]
    for l in range(16):
      k = m * 16 + l
      wb = jnp.full((16,), w16[l], jnp.float32)
      for d in range(D // 16):
        rows_v[k, pl.ds(d * 16, 16)] = rows_v[k, pl.ds(d * 16, 16)] * wb
    return 0
  lax.fori_loop(0, K // 16, scale, 0)


def _zero_acc(rows0, acc_sh, sid, D):
  def zslab(i, _):
    for d in range(D // 16):
      rows0[i, pl.ds(d * 16, 16)] = jnp.zeros((16,), jnp.float32)
    return 0
  lax.fori_loop(0, K, zslab, 0)

  def zacc(i, _):
    pltpu.sync_copy(rows0, acc_sh.at[pl.ds(sid * TB + i * K, K)])
    return 0
  lax.fori_loop(0, TROWS // K, zacc, 0)


def _copy_out(acc_sh, out_hbm, cid, sid):
  pltpu.sync_copy(acc_sh.at[pl.ds(sid * TB, TROWS)],
                  out_hbm.at[cid, pl.ds(sid * TB, TROWS)])


def _make_lap_sc(D):
  def body(y_hbm, w2_hbm, row2_hbm, col2_hbm, out_hbm,
           ridx_a, cidx_a, w_a, rows0, rows1, acc_sh,
           gsem0, gsem1, ssem0, ssem1):
    cid = lax.axis_index("c")
    sid = lax.axis_index("s")
    wid = sid * NC + cid

    _zero_acc(rows0, acc_sh, sid, D)
    plsc.subcore_barrier()

    rows = (rows0, rows1)
    gsems = (gsem0, gsem1)
    ssems = (ssem0, ssem1)

    def stage(s, _):
      base = wid * NCH + s * IB
      pltpu.sync_copy(row2_hbm.at[pl.ds(base, IB)], ridx_a)
      pltpu.sync_copy(col2_hbm.at[pl.ds(base, IB)], cidx_a)
      pltpu.sync_copy(w2_hbm.at[pl.ds(base, IB)], w_a)

      pltpu.async_copy(y_hbm.at[ridx_a.at[0]], rows0, gsem0)

      def outer(i, _):
        for b in range(2):
          jj = i * 2 + b
          bo = 1 - b

          @pl.when(jj > 0)
          def _():
            pltpu.make_async_copy(rows[bo], acc_sh.at[cidx_a.at[jj - 1]],
                                  ssems[bo]).wait()

          @pl.when(jj + 1 < IB)
          def _():
            pltpu.async_copy(y_hbm.at[ridx_a.at[jj + 1]], rows[bo],
                             gsems[bo])

          pltpu.make_async_copy(y_hbm.at[ridx_a.at[jj]], rows[b],
                                gsems[b]).wait()
          _scale_chunk(rows[b], w_a.at[jj], D)
          pltpu.async_copy(rows[b], acc_sh.at[cidx_a.at[jj]], ssems[b],
                           add=True)
        return 0
      lax.fori_loop(0, IB // 2, outer, 0)

      pltpu.make_async_copy(rows1, acc_sh.at[cidx_a.at[IB - 1]],
                            ssem1).wait()
      return 0
    lax.fori_loop(0, NSTG, stage, 0)

    plsc.subcore_barrier()
    _copy_out(acc_sh, out_hbm, cid, sid)

  return functools.partial(
      pl.kernel,
      mesh=_sc_mesh(),
      out_type=jax.ShapeDtypeStruct((NC, N, D), jnp.float32),
      scratch_types=[
          pltpu.VMEM((IB, K), jnp.int32),
          pltpu.VMEM((IB, K), jnp.int32),
          pltpu.VMEM((IB, K), jnp.float32),
          pltpu.VMEM((K, D), jnp.float32),
          pltpu.VMEM((K, D), jnp.float32),
          pltpu.VMEM_SHARED((N, D), jnp.float32),
          pltpu.SemaphoreType.DMA,
          pltpu.SemaphoreType.DMA,
          pltpu.SemaphoreType.DMA,
          pltpu.SemaphoreType.DMA,
      ],
  )(body)


def _make_deg_sc():
  D = 128

  def body(w2_hbm, row2_hbm, out_hbm,
           ridx_a, w_a, rows0, rows1, acc_sh, ssem0, ssem1):
    cid = lax.axis_index("c")
    sid = lax.axis_index("s")
    wid = sid * NC + cid

    _zero_acc(rows0, acc_sh, sid, D)
    plsc.subcore_barrier()

    rows = (rows0, rows1)
    ssems = (ssem0, ssem1)

    def fill(buf, w_row):
      def gen(m, _):
        w16 = w_row[pl.ds(m * 16, 16)]
        for l in range(16):
          wb = jnp.full((16,), w16[l], jnp.float32)
          for d in range(D // 16):
            buf[m * 16 + l, pl.ds(d * 16, 16)] = wb
        return 0
      lax.fori_loop(0, K // 16, gen, 0)

    def stage(s, _):
      base = wid * NCH + s * IB
      pltpu.sync_copy(row2_hbm.at[pl.ds(base, IB)], ridx_a)
      pltpu.sync_copy(w2_hbm.at[pl.ds(base, IB)], w_a)

      def outer(i, _):
        for b in range(2):
          jj = i * 2 + b

          @pl.when(jj > 1)
          def _():
            pltpu.make_async_copy(rows[b], acc_sh.at[ridx_a.at[jj - 2]],
                                  ssems[b]).wait()

          fill(rows[b], w_a.at[jj])
          pltpu.async_copy(rows[b], acc_sh.at[ridx_a.at[jj]], ssems[b],
                           add=True)
        return 0
      lax.fori_loop(0, IB // 2, outer, 0)

      pltpu.make_async_copy(rows0, acc_sh.at[ridx_a.at[IB - 2]],
                            ssem0).wait()
      pltpu.make_async_copy(rows1, acc_sh.at[ridx_a.at[IB - 1]],
                            ssem1).wait()
      return 0
    lax.fori_loop(0, NSTG, stage, 0)

    plsc.subcore_barrier()
    _copy_out(acc_sh, out_hbm, cid, sid)

  return functools.partial(
      pl.kernel,
      mesh=_sc_mesh(),
      out_type=jax.ShapeDtypeStruct((NC, N, D), jnp.float32),
      scratch_types=[
          pltpu.VMEM((IB, K), jnp.int32),
          pltpu.VMEM((IB, K), jnp.float32),
          pltpu.VMEM((K, D), jnp.float32),
          pltpu.VMEM((K, D), jnp.float32),
          pltpu.VMEM_SHARED((N, D), jnp.float32),
          pltpu.SemaphoreType.DMA,
          pltpu.SemaphoreType.DMA,
      ],
  )(body)


BN = 2000


def _tc_pre(degp_ref, x_ref, dis_ref, y1_ref):
  d16 = degp_ref[0, :, :16] + degp_ref[1, :, :16]
  dis = jnp.where(d16 > 0, lax.rsqrt(jnp.maximum(d16, 1e-12)),
                  jnp.zeros_like(d16))
  dis_ref[...] = dis
  y1_ref[...] = dis[:, :1] * x_ref[...]


def _tc_cell1(x_ref, aggp_ref, dis_ref, w0_ref, w1_ref, b_ref, wc2_ref,
              h1_ref, y2_ref):
  dis = dis_ref[:, :1]
  lx = (-dis) * (aggp_ref[0] + aggp_ref[1])
  g = (jnp.dot(x_ref[...], w0_ref[...], preferred_element_type=jnp.float32)
       + jnp.dot(lx, w1_ref[...], preferred_element_type=jnp.float32)
       + b_ref[...])
  i = jax.nn.sigmoid(g[:, :H1])
  t = jnp.tanh(g[:, H1:2 * H1])
  c = i * t
  o = jax.nn.sigmoid(g[:, 2 * H1:3 * H1] + wc2_ref[...] * c)
  h1 = jax.nn.relu(o * jnp.tanh(c))
  h1p = jnp.concatenate(
      [h1, jnp.zeros((h1.shape[0], H2PAD - H1), h1.dtype)], axis=1)
  h1_ref[...] = h1p
  y2_ref[...] = dis * h1p


def _tc_cell2(h1_ref, aggp_ref, dis_ref, w0_ref, w1_ref, b_ref, wc2_ref,
              lw_ref, lb_ref, out_ref):
  dis = dis_ref[:, :1]
  lx = (-dis) * (aggp_ref[0] + aggp_ref[1])
  g = (jnp.dot(h1_ref[...], w0_ref[...], preferred_element_type=jnp.float32)
       + jnp.dot(lx, w1_ref[...], preferred_element_type=jnp.float32)
       + b_ref[...])
  i = jax.nn.sigmoid(g[:, :H2])
  t = jnp.tanh(g[:, H2:2 * H2])
  c = i * t
  o = jax.nn.sigmoid(g[:, 2 * H2:3 * H2] + wc2_ref[...] * c)
  h2 = jax.nn.relu(o * jnp.tanh(c))
  out_ref[...] = (jnp.dot(h2, lw_ref[...], preferred_element_type=jnp.float32)
                  + lb_ref[...])


def _row_spec(w):
  return pl.BlockSpec((BN, w), lambda i: (i, 0))


def _full_spec(shape):
  return pl.BlockSpec(shape, lambda i: tuple(0 for _ in shape))


def _aggp_spec(w):
  return pl.BlockSpec((NC, BN, w), lambda i: (0, i, 0))


_GRID = (N // BN,)


def kernel(x, edge_index, edge_weight, l1_W0x, l1_W1x, l1_bx, l1_W0h, l1_W1h,
           l1_bh, l1_wc, l1_bg, l2_W0x, l2_W1x, l2_bx, l2_W0h, l2_W1h, l2_bh,
           l2_wc, l2_bg, lin_W, lin_b):
  pad = EPAD - E
  row2 = jnp.concatenate([edge_index[0], jnp.zeros((pad,), jnp.int32)])
  row2 = row2.reshape(NW * NCH, K)
  col2 = jnp.concatenate([edge_index[1], jnp.zeros((pad,), jnp.int32)])
  col2 = col2.reshape(NW * NCH, K)
  w2 = jnp.concatenate([edge_weight, jnp.zeros((pad,), jnp.float32)])
  w2 = w2.reshape(NW * NCH, K)

  gsel = jnp.array([0, 2, 3], jnp.int32)
  w0c1 = jnp.concatenate([l1_W0x[g] for g in (0, 2, 3)], axis=1)
  w1c1 = jnp.concatenate([l1_W1x[g] for g in (0, 2, 3)], axis=1)
  b1 = (l1_bx + l1_bh + l1_bg)[gsel].reshape(1, 3 * H1)
  wc21 = l1_wc[2].reshape(1, H1)

  zpad = jnp.zeros((H2PAD - H1, 3 * H2), jnp.float32)
  w0c2 = jnp.concatenate(
      [jnp.concatenate([l2_W0x[g] for g in (0, 2, 3)], axis=1), zpad], axis=0)
  w1c2 = jnp.concatenate(
      [jnp.concatenate([l2_W1x[g] for g in (0, 2, 3)], axis=1), zpad], axis=0)
  b2 = (l2_bx + l2_bh + l2_bg)[gsel].reshape(1, 3 * H2)
  wc22 = l2_wc[2].reshape(1, H2)
  linb = lin_b.reshape(1, 1)

  degp = _make_deg_sc()(w2, row2)

  dis16, y1 = pl.pallas_call(
      _tc_pre,
      grid=_GRID,
      in_specs=[_aggp_spec(D_IN), _row_spec(D_IN)],
      out_specs=[_row_spec(16), _row_spec(D_IN)],
      out_shape=[jax.ShapeDtypeStruct((N, 16), jnp.float32),
                 jax.ShapeDtypeStruct((N, D_IN), jnp.float32)],
  )(degp, x)

  agg1 = _make_lap_sc(D_IN)(y1, w2, row2, col2)

  h1p, y2 = pl.pallas_call(
      _tc_cell1,
      grid=_GRID,
      in_specs=[_row_spec(D_IN), _aggp_spec(D_IN), _row_spec(16),
                _full_spec((D_IN, 3 * H1)), _full_spec((D_IN, 3 * H1)),
                _full_spec((1, 3 * H1)), _full_spec((1, H1))],
      out_specs=[_row_spec(H2PAD), _row_spec(H2PAD)],
      out_shape=[jax.ShapeDtypeStruct((N, H2PAD), jnp.float32),
                 jax.ShapeDtypeStruct((N, H2PAD), jnp.float32)],
  )(x, agg1, dis16, w0c1, w1c1, b1, wc21)

  agg2 = _make_lap_sc(H2PAD)(y2, w2, row2, col2)

  out = pl.pallas_call(
      _tc_cell2,
      grid=_GRID,
      in_specs=[_row_spec(H2PAD), _aggp_spec(H2PAD), _row_spec(16),
                _full_spec((H2PAD, 3 * H2)), _full_spec((H2PAD, 3 * H2)),
                _full_spec((1, 3 * H2)), _full_spec((1, H2)),
                _full_spec((H2, 1)), _full_spec((1, 1))],
      out_specs=_row_spec(1),
      out_shape=jax.ShapeDtypeStruct((N, 1), jnp.float32),
  )(h1p, agg2, dis16, w0c2, w1c2, b2, wc22, lin_W, linb)

  return out


# speedup vs baseline: 8.9672x; 1.0005x over previous
import functools

import jax
import jax.numpy as jnp
from jax import lax
from jax.experimental import pallas as pl
from jax.experimental.pallas import tpu as pltpu
from jax.experimental.pallas import tpu_sc as plsc

N = 10000
E = 320000
D_IN = 128
H1 = 50
H2 = 20
H2PAD = 128

NC = 2
NS = 16
NW = NC * NS
K = 128
NCH = 80
IB = 16
NSTG = NCH // IB
EPAD = NW * NCH * K
TB = 624
TROWS = 640

def _sc_mesh():
  return plsc.VectorSubcoreMesh(core_axis_name="c", subcore_axis_name="s")


def _scale_chunk(rows_v, w_row, D):
  def scale(m, _):
    w16 = w_row[pl.ds(m * 16, 16)]
    for l in range(16):
      k = m * 16 + l
      wb = jnp.full((16,), w16[l], jnp.float32)
      for d in range(D // 16):
        rows_v[k, pl.ds(d * 16, 16)] = rows_v[k, pl.ds(d * 16, 16)] * wb
    return 0
  lax.fori_loop(0, K // 16, scale, 0)


def _zero_acc(rows0, acc_sh, sid, D):
  def zslab(i, _):
    for d in range(D // 16):
      rows0[i, pl.ds(d * 16, 16)] = jnp.zeros((16,), jnp.float32)
    return 0
  lax.fori_loop(0, K, zslab, 0)

  def zacc(i, _):
    pltpu.sync_copy(rows0, acc_sh.at[pl.ds(sid * TB + i * K, K)])
    return 0
  lax.fori_loop(0, TROWS // K, zacc, 0)


def _copy_out(acc_sh, out_hbm, cid, sid):
  pltpu.sync_copy(acc_sh.at[pl.ds(sid * TB, TROWS)],
                  out_hbm.at[cid, pl.ds(sid * TB, TROWS)])


def _make_lap_sc(D):
  def body(y_hbm, w2_hbm, row2_hbm, col2_hbm, out_hbm,
           ridx_a, cidx_a, w_a, rows0, rows1, acc_sh,
           gsem0, gsem1, ssem0, ssem1):
    cid = lax.axis_index("c")
    sid = lax.axis_index("s")
    wid = sid * NC + cid

    _zero_acc(rows0, acc_sh, sid, D)
    plsc.subcore_barrier()

    rows = (rows0, rows1)
    gsems = (gsem0, gsem1)
    ssems = (ssem0, ssem1)

    def stage(s, _):
      base = wid * NCH + s * IB
      pltpu.sync_copy(row2_hbm.at[pl.ds(base, IB)], ridx_a)
      pltpu.sync_copy(col2_hbm.at[pl.ds(base, IB)], cidx_a)
      pltpu.sync_copy(w2_hbm.at[pl.ds(base, IB)], w_a)

      pltpu.async_copy(y_hbm.at[ridx_a.at[0]], rows0, gsem0)

      def outer(i, _):
        for b in range(2):
          jj = i * 2 + b
          bo = 1 - b

          @pl.when(jj + 1 < IB)
          def _():
            pltpu.async_copy(y_hbm.at[ridx_a.at[jj + 1]], rows[bo],
                             gsems[bo])

          pltpu.make_async_copy(y_hbm.at[ridx_a.at[jj]], rows[b],
                                gsems[b]).wait()
          _scale_chunk(rows[b], w_a.at[jj], D)
          pltpu.sync_copy(rows[b], acc_sh.at[cidx_a.at[jj]], add=True)
        return 0
      lax.fori_loop(0, IB // 2, outer, 0)
      return 0
    lax.fori_loop(0, NSTG, stage, 0)

    plsc.subcore_barrier()
    _copy_out(acc_sh, out_hbm, cid, sid)

  return functools.partial(
      pl.kernel,
      mesh=_sc_mesh(),
      out_type=jax.ShapeDtypeStruct((NC, N, D), jnp.float32),
      scratch_types=[
          pltpu.VMEM((IB, K), jnp.int32),
          pltpu.VMEM((IB, K), jnp.int32),
          pltpu.VMEM((IB, K), jnp.float32),
          pltpu.VMEM((K, D), jnp.float32),
          pltpu.VMEM((K, D), jnp.float32),
          pltpu.VMEM_SHARED((N, D), jnp.float32),
          pltpu.SemaphoreType.DMA,
          pltpu.SemaphoreType.DMA,
          pltpu.SemaphoreType.DMA,
          pltpu.SemaphoreType.DMA,
      ],
  )(body)


def _make_deg_sc():
  D = 128

  def body(w2_hbm, row2_hbm, out_hbm,
           ridx_a, w_a, rows0, rows1, acc_sh, ssem0, ssem1):
    cid = lax.axis_index("c")
    sid = lax.axis_index("s")
    wid = sid * NC + cid

    _zero_acc(rows0, acc_sh, sid, D)
    plsc.subcore_barrier()

    rows = (rows0, rows1)
    ssems = (ssem0, ssem1)

    def fill(buf, w_row):
      def gen(m, _):
        w16 = w_row[pl.ds(m * 16, 16)]
        for l in range(16):
          wb = jnp.full((16,), w16[l], jnp.float32)
          for d in range(D // 16):
            buf[m * 16 + l, pl.ds(d * 16, 16)] = wb
        return 0
      lax.fori_loop(0, K // 16, gen, 0)

    def stage(s, _):
      base = wid * NCH + s * IB
      pltpu.sync_copy(row2_hbm.at[pl.ds(base, IB)], ridx_a)
      pltpu.sync_copy(w2_hbm.at[pl.ds(base, IB)], w_a)

      def outer(i, _):
        for b in range(2):
          jj = i * 2 + b

          @pl.when(jj > 1)
          def _():
            pltpu.make_async_copy(rows[b], acc_sh.at[ridx_a.at[jj - 2]],
                                  ssems[b]).wait()

          fill(rows[b], w_a.at[jj])
          pltpu.async_copy(rows[b], acc_sh.at[ridx_a.at[jj]], ssems[b],
                           add=True)
        return 0
      lax.fori_loop(0, IB // 2, outer, 0)

      pltpu.make_async_copy(rows0, acc_sh.at[ridx_a.at[IB - 2]],
                            ssem0).wait()
      pltpu.make_async_copy(rows1, acc_sh.at[ridx_a.at[IB - 1]],
                            ssem1).wait()
      return 0
    lax.fori_loop(0, NSTG, stage, 0)

    plsc.subcore_barrier()
    _copy_out(acc_sh, out_hbm, cid, sid)

  return functools.partial(
      pl.kernel,
      mesh=_sc_mesh(),
      out_type=jax.ShapeDtypeStruct((NC, N, D), jnp.float32),
      scratch_types=[
          pltpu.VMEM((IB, K), jnp.int32),
          pltpu.VMEM((IB, K), jnp.float32),
          pltpu.VMEM((K, D), jnp.float32),
          pltpu.VMEM((K, D), jnp.float32),
          pltpu.VMEM_SHARED((N, D), jnp.float32),
          pltpu.SemaphoreType.DMA,
          pltpu.SemaphoreType.DMA,
      ],
  )(body)


BN = 2000


def _tc_pre(degp_ref, x_ref, dis_ref, y1_ref):
  d16 = degp_ref[0, :, :16] + degp_ref[1, :, :16]
  dis = jnp.where(d16 > 0, lax.rsqrt(jnp.maximum(d16, 1e-12)),
                  jnp.zeros_like(d16))
  dis_ref[...] = dis
  y1_ref[...] = dis[:, :1] * x_ref[...]


def _tc_cell1(x_ref, aggp_ref, dis_ref, w0_ref, w1_ref, b_ref, wc2_ref,
              h1_ref, y2_ref):
  dis = dis_ref[:, :1]
  lx = (-dis) * (aggp_ref[0] + aggp_ref[1])
  g = (jnp.dot(x_ref[...], w0_ref[...], preferred_element_type=jnp.float32)
       + jnp.dot(lx, w1_ref[...], preferred_element_type=jnp.float32)
       + b_ref[...])
  i = jax.nn.sigmoid(g[:, :H1])
  t = jnp.tanh(g[:, H1:2 * H1])
  c = i * t
  o = jax.nn.sigmoid(g[:, 2 * H1:3 * H1] + wc2_ref[...] * c)
  h1 = jax.nn.relu(o * jnp.tanh(c))
  h1p = jnp.concatenate(
      [h1, jnp.zeros((h1.shape[0], H2PAD - H1), h1.dtype)], axis=1)
  h1_ref[...] = h1p
  y2_ref[...] = dis * h1p


def _tc_cell2(h1_ref, aggp_ref, dis_ref, w0_ref, w1_ref, b_ref, wc2_ref,
              lw_ref, lb_ref, out_ref):
  dis = dis_ref[:, :1]
  lx = (-dis) * (aggp_ref[0] + aggp_ref[1])
  g = (jnp.dot(h1_ref[...], w0_ref[...], preferred_element_type=jnp.float32)
       + jnp.dot(lx, w1_ref[...], preferred_element_type=jnp.float32)
       + b_ref[...])
  i = jax.nn.sigmoid(g[:, :H2])
  t = jnp.tanh(g[:, H2:2 * H2])
  c = i * t
  o = jax.nn.sigmoid(g[:, 2 * H2:3 * H2] + wc2_ref[...] * c)
  h2 = jax.nn.relu(o * jnp.tanh(c))
  out_ref[...] = (jnp.dot(h2, lw_ref[...], preferred_element_type=jnp.float32)
                  + lb_ref[...])


def _row_spec(w):
  return pl.BlockSpec((BN, w), lambda i: (i, 0))


def _full_spec(shape):
  return pl.BlockSpec(shape, lambda i: tuple(0 for _ in shape))


def _aggp_spec(w):
  return pl.BlockSpec((NC, BN, w), lambda i: (0, i, 0))


_GRID = (N // BN,)


def kernel(x, edge_index, edge_weight, l1_W0x, l1_W1x, l1_bx, l1_W0h, l1_W1h,
           l1_bh, l1_wc, l1_bg, l2_W0x, l2_W1x, l2_bx, l2_W0h, l2_W1h, l2_bh,
           l2_wc, l2_bg, lin_W, lin_b):
  pad = EPAD - E
  row2 = jnp.concatenate([edge_index[0], jnp.zeros((pad,), jnp.int32)])
  row2 = row2.reshape(NW * NCH, K)
  col2 = jnp.concatenate([edge_index[1], jnp.zeros((pad,), jnp.int32)])
  col2 = col2.reshape(NW * NCH, K)
  w2 = jnp.concatenate([edge_weight, jnp.zeros((pad,), jnp.float32)])
  w2 = w2.reshape(NW * NCH, K)

  gsel = jnp.array([0, 2, 3], jnp.int32)
  w0c1 = jnp.concatenate([l1_W0x[g] for g in (0, 2, 3)], axis=1)
  w1c1 = jnp.concatenate([l1_W1x[g] for g in (0, 2, 3)], axis=1)
  b1 = (l1_bx + l1_bh + l1_bg)[gsel].reshape(1, 3 * H1)
  wc21 = l1_wc[2].reshape(1, H1)

  zpad = jnp.zeros((H2PAD - H1, 3 * H2), jnp.float32)
  w0c2 = jnp.concatenate(
      [jnp.concatenate([l2_W0x[g] for g in (0, 2, 3)], axis=1), zpad], axis=0)
  w1c2 = jnp.concatenate(
      [jnp.concatenate([l2_W1x[g] for g in (0, 2, 3)], axis=1), zpad], axis=0)
  b2 = (l2_bx + l2_bh + l2_bg)[gsel].reshape(1, 3 * H2)
  wc22 = l2_wc[2].reshape(1, H2)
  linb = lin_b.reshape(1, 1)

  degp = _make_deg_sc()(w2, row2)

  dis16, y1 = pl.pallas_call(
      _tc_pre,
      grid=_GRID,
      in_specs=[_aggp_spec(D_IN), _row_spec(D_IN)],
      out_specs=[_row_spec(16), _row_spec(D_IN)],
      out_shape=[jax.ShapeDtypeStruct((N, 16), jnp.float32),
                 jax.ShapeDtypeStruct((N, D_IN), jnp.float32)],
  )(degp, x)

  agg1 = _make_lap_sc(D_IN)(y1, w2, row2, col2)

  h1p, y2 = pl.pallas_call(
      _tc_cell1,
      grid=_GRID,
      in_specs=[_row_spec(D_IN), _aggp_spec(D_IN), _row_spec(16),
                _full_spec((D_IN, 3 * H1)), _full_spec((D_IN, 3 * H1)),
                _full_spec((1, 3 * H1)), _full_spec((1, H1))],
      out_specs=[_row_spec(H2PAD), _row_spec(H2PAD)],
      out_shape=[jax.ShapeDtypeStruct((N, H2PAD), jnp.float32),
                 jax.ShapeDtypeStruct((N, H2PAD), jnp.float32)],
  )(x, agg1, dis16, w0c1, w1c1, b1, wc21)

  agg2 = _make_lap_sc(H2PAD)(y2, w2, row2, col2)

  out = pl.pallas_call(
      _tc_cell2,
      grid=_GRID,
      in_specs=[_row_spec(H2PAD), _aggp_spec(H2PAD), _row_spec(16),
                _full_spec((H2PAD, 3 * H2)), _full_spec((H2PAD, 3 * H2)),
                _full_spec((1, 3 * H2)), _full_spec((1, H2)),
                _full_spec((H2, 1)), _full_spec((1, 1))],
      out_specs=_row_spec(1),
      out_shape=jax.ShapeDtypeStruct((N, 1), jnp.float32),
  )(h1p, agg2, dis16, w0c2, w1c2, b2, wc22, lin_W, linb)

  return out


# speedup vs baseline: 10.4834x; 1.1691x over previous
import functools

import jax
import jax.numpy as jnp
from jax import lax
from jax.experimental import pallas as pl
from jax.experimental.pallas import tpu as pltpu
from jax.experimental.pallas import tpu_sc as plsc

N = 10000
E = 320000
D_IN = 128
H1 = 50
H2 = 20
H2PAD = 128

NC = 2
NS = 16
NW = NC * NS
K = 128
NCH = 80
IB = 16
NSTG = NCH // IB
NCH0 = 120
IBL = 8
EPAD = NW * NCH * K
TB = 624
TROWS = 640

def _sc_mesh():
  return plsc.VectorSubcoreMesh(core_axis_name="c", subcore_axis_name="s")


def _scale_chunk(rows_v, w_row, D):
  def scale(m, _):
    w16 = w_row[pl.ds(m * 16, 16)]
    for l in range(16):
      k = m * 16 + l
      wb = jnp.full((16,), w16[l], jnp.float32)
      for d in range(D // 16):
        rows_v[k, pl.ds(d * 16, 16)] = rows_v[k, pl.ds(d * 16, 16)] * wb
    return 0
  lax.fori_loop(0, K // 16, scale, 0)


def _zero_acc(rows0, acc_sh, sid, D):
  def zslab(i, _):
    for d in range(D // 16):
      rows0[i, pl.ds(d * 16, 16)] = jnp.zeros((16,), jnp.float32)
    return 0
  lax.fori_loop(0, K, zslab, 0)

  def zacc(i, _):
    pltpu.sync_copy(rows0, acc_sh.at[pl.ds(sid * TB + i * K, K)])
    return 0
  lax.fori_loop(0, TROWS // K, zacc, 0)


def _copy_out(acc_sh, out_hbm, cid, sid):
  pltpu.sync_copy(acc_sh.at[pl.ds(sid * TB, TROWS)],
                  out_hbm.at[cid, pl.ds(sid * TB, TROWS)])


def _make_lap_sc(D):
  def body(y_hbm, w2_hbm, row2_hbm, col2_hbm, out_hbm,
           ridx_a, cidx_a, w_a, rows0, rows1, acc_sh,
           gsem0, gsem1, ssem0, ssem1):
    cid = lax.axis_index("c")
    sid = lax.axis_index("s")

    _zero_acc(rows0, acc_sh, sid, D)
    plsc.subcore_barrier()

    rows = (rows0, rows1)
    gsems = (gsem0, gsem1)

    base0 = sid * (2 * NCH) + cid * NCH0
    nstg = lax.select(cid == 0, NCH0 // IBL, (2 * NCH - NCH0) // IBL)

    def stage(s, _):
      base = base0 + s * IBL
      pltpu.sync_copy(row2_hbm.at[pl.ds(base, IBL)], ridx_a)
      pltpu.sync_copy(col2_hbm.at[pl.ds(base, IBL)], cidx_a)
      pltpu.sync_copy(w2_hbm.at[pl.ds(base, IBL)], w_a)

      pltpu.async_copy(y_hbm.at[ridx_a.at[0]], rows0, gsem0)

      def outer(i, _):
        for b in range(2):
          jj = i * 2 + b
          bo = 1 - b

          @pl.when(jj + 1 < IBL)
          def _():
            pltpu.async_copy(y_hbm.at[ridx_a.at[jj + 1]], rows[bo],
                             gsems[bo])

          pltpu.make_async_copy(y_hbm.at[ridx_a.at[jj]], rows[b],
                                gsems[b]).wait()
          _scale_chunk(rows[b], w_a.at[jj], D)
          pltpu.sync_copy(rows[b], acc_sh.at[cidx_a.at[jj]], add=True)
        return 0
      lax.fori_loop(0, IBL // 2, outer, 0)
      return 0
    lax.fori_loop(0, nstg, stage, 0)

    plsc.subcore_barrier()
    _copy_out(acc_sh, out_hbm, cid, sid)

  return functools.partial(
      pl.kernel,
      mesh=_sc_mesh(),
      out_type=jax.ShapeDtypeStruct((NC, N, D), jnp.float32),
      scratch_types=[
          pltpu.VMEM((IBL, K), jnp.int32),
          pltpu.VMEM((IBL, K), jnp.int32),
          pltpu.VMEM((IBL, K), jnp.float32),
          pltpu.VMEM((K, D), jnp.float32),
          pltpu.VMEM((K, D), jnp.float32),
          pltpu.VMEM_SHARED((N, D), jnp.float32),
          pltpu.SemaphoreType.DMA,
          pltpu.SemaphoreType.DMA,
          pltpu.SemaphoreType.DMA,
          pltpu.SemaphoreType.DMA,
      ],
  )(body)


def _make_deg_sc():
  D = 128

  def body(w2_hbm, row2_hbm, out_hbm,
           ridx_a, w_a, rows0, rows1, acc_sh, ssem0, ssem1):
    cid = lax.axis_index("c")
    sid = lax.axis_index("s")
    wid = sid * NC + cid

    _zero_acc(rows0, acc_sh, sid, D)
    plsc.subcore_barrier()

    rows = (rows0, rows1)
    ssems = (ssem0, ssem1)

    def fill(buf, w_row):
      def gen(m, _):
        w16 = w_row[pl.ds(m * 16, 16)]
        for l in range(16):
          wb = jnp.full((16,), w16[l], jnp.float32)
          for d in range(D // 16):
            buf[m * 16 + l, pl.ds(d * 16, 16)] = wb
        return 0
      lax.fori_loop(0, K // 16, gen, 0)

    def stage(s, _):
      base = wid * NCH + s * IB
      pltpu.sync_copy(row2_hbm.at[pl.ds(base, IB)], ridx_a)
      pltpu.sync_copy(w2_hbm.at[pl.ds(base, IB)], w_a)

      def outer(i, _):
        for b in range(2):
          jj = i * 2 + b

          @pl.when(jj > 1)
          def _():
            pltpu.make_async_copy(rows[b], acc_sh.at[ridx_a.at[jj - 2]],
                                  ssems[b]).wait()

          fill(rows[b], w_a.at[jj])
          pltpu.async_copy(rows[b], acc_sh.at[ridx_a.at[jj]], ssems[b],
                           add=True)
        return 0
      lax.fori_loop(0, IB // 2, outer, 0)

      pltpu.make_async_copy(rows0, acc_sh.at[ridx_a.at[IB - 2]],
                            ssem0).wait()
      pltpu.make_async_copy(rows1, acc_sh.at[ridx_a.at[IB - 1]],
                            ssem1).wait()
      return 0
    lax.fori_loop(0, NSTG, stage, 0)

    plsc.subcore_barrier()
    _copy_out(acc_sh, out_hbm, cid, sid)

  return functools.partial(
      pl.kernel,
      mesh=_sc_mesh(),
      out_type=jax.ShapeDtypeStruct((NC, N, D), jnp.float32),
      scratch_types=[
          pltpu.VMEM((IB, K), jnp.int32),
          pltpu.VMEM((IB, K), jnp.float32),
          pltpu.VMEM((K, D), jnp.float32),
          pltpu.VMEM((K, D), jnp.float32),
          pltpu.VMEM_SHARED((N, D), jnp.float32),
          pltpu.SemaphoreType.DMA,
          pltpu.SemaphoreType.DMA,
      ],
  )(body)


BN = 2000


def _tc_pre(degp_ref, x_ref, dis_ref, y1_ref):
  d16 = degp_ref[0, :, :16] + degp_ref[1, :, :16]
  dis = jnp.where(d16 > 0, lax.rsqrt(jnp.maximum(d16, 1e-12)),
                  jnp.zeros_like(d16))
  dis_ref[...] = dis
  y1_ref[...] = dis[:, :1] * x_ref[...]


def _tc_cell1(x_ref, aggp_ref, dis_ref, w0_ref, w1_ref, b_ref, wc2_ref,
              h1_ref, y2_ref):
  dis = dis_ref[:, :1]
  lx = (-dis) * (aggp_ref[0] + aggp_ref[1])
  g = (jnp.dot(x_ref[...], w0_ref[...], preferred_element_type=jnp.float32)
       + jnp.dot(lx, w1_ref[...], preferred_element_type=jnp.float32)
       + b_ref[...])
  i = jax.nn.sigmoid(g[:, :H1])
  t = jnp.tanh(g[:, H1:2 * H1])
  c = i * t
  o = jax.nn.sigmoid(g[:, 2 * H1:3 * H1] + wc2_ref[...] * c)
  h1 = jax.nn.relu(o * jnp.tanh(c))
  h1p = jnp.concatenate(
      [h1, jnp.zeros((h1.shape[0], H2PAD - H1), h1.dtype)], axis=1)
  h1_ref[...] = h1p
  y2_ref[...] = dis * h1p


def _tc_cell2(h1_ref, aggp_ref, dis_ref, w0_ref, w1_ref, b_ref, wc2_ref,
              lw_ref, lb_ref, out_ref):
  dis = dis_ref[:, :1]
  lx = (-dis) * (aggp_ref[0] + aggp_ref[1])
  g = (jnp.dot(h1_ref[...], w0_ref[...], preferred_element_type=jnp.float32)
       + jnp.dot(lx, w1_ref[...], preferred_element_type=jnp.float32)
       + b_ref[...])
  i = jax.nn.sigmoid(g[:, :H2])
  t = jnp.tanh(g[:, H2:2 * H2])
  c = i * t
  o = jax.nn.sigmoid(g[:, 2 * H2:3 * H2] + wc2_ref[...] * c)
  h2 = jax.nn.relu(o * jnp.tanh(c))
  out_ref[...] = (jnp.dot(h2, lw_ref[...], preferred_element_type=jnp.float32)
                  + lb_ref[...])


def _row_spec(w):
  return pl.BlockSpec((BN, w), lambda i: (i, 0))


def _full_spec(shape):
  return pl.BlockSpec(shape, lambda i: tuple(0 for _ in shape))


def _aggp_spec(w):
  return pl.BlockSpec((NC, BN, w), lambda i: (0, i, 0))


_GRID = (N // BN,)


def kernel(x, edge_index, edge_weight, l1_W0x, l1_W1x, l1_bx, l1_W0h, l1_W1h,
           l1_bh, l1_wc, l1_bg, l2_W0x, l2_W1x, l2_bx, l2_W0h, l2_W1h, l2_bh,
           l2_wc, l2_bg, lin_W, lin_b):
  pad = EPAD - E
  row2 = jnp.concatenate([edge_index[0], jnp.zeros((pad,), jnp.int32)])
  row2 = row2.reshape(NW * NCH, K)
  col2 = jnp.concatenate([edge_index[1], jnp.zeros((pad,), jnp.int32)])
  col2 = col2.reshape(NW * NCH, K)
  w2 = jnp.concatenate([edge_weight, jnp.zeros((pad,), jnp.float32)])
  w2 = w2.reshape(NW * NCH, K)

  gsel = jnp.array([0, 2, 3], jnp.int32)
  w0c1 = jnp.concatenate([l1_W0x[g] for g in (0, 2, 3)], axis=1)
  w1c1 = jnp.concatenate([l1_W1x[g] for g in (0, 2, 3)], axis=1)
  b1 = (l1_bx + l1_bh + l1_bg)[gsel].reshape(1, 3 * H1)
  wc21 = l1_wc[2].reshape(1, H1)

  zpad = jnp.zeros((H2PAD - H1, 3 * H2), jnp.float32)
  w0c2 = jnp.concatenate(
      [jnp.concatenate([l2_W0x[g] for g in (0, 2, 3)], axis=1), zpad], axis=0)
  w1c2 = jnp.concatenate(
      [jnp.concatenate([l2_W1x[g] for g in (0, 2, 3)], axis=1), zpad], axis=0)
  b2 = (l2_bx + l2_bh + l2_bg)[gsel].reshape(1, 3 * H2)
  wc22 = l2_wc[2].reshape(1, H2)
  linb = lin_b.reshape(1, 1)

  degp = _make_deg_sc()(w2, row2)

  dis16, y1 = pl.pallas_call(
      _tc_pre,
      grid=_GRID,
      in_specs=[_aggp_spec(D_IN), _row_spec(D_IN)],
      out_specs=[_row_spec(16), _row_spec(D_IN)],
      out_shape=[jax.ShapeDtypeStruct((N, 16), jnp.float32),
                 jax.ShapeDtypeStruct((N, D_IN), jnp.float32)],
  )(degp, x)

  agg1 = _make_lap_sc(D_IN)(y1, w2, row2, col2)

  h1p, y2 = pl.pallas_call(
      _tc_cell1,
      grid=_GRID,
      in_specs=[_row_spec(D_IN), _aggp_spec(D_IN), _row_spec(16),
                _full_spec((D_IN, 3 * H1)), _full_spec((D_IN, 3 * H1)),
                _full_spec((1, 3 * H1)), _full_spec((1, H1))],
      out_specs=[_row_spec(H2PAD), _row_spec(H2PAD)],
      out_shape=[jax.ShapeDtypeStruct((N, H2PAD), jnp.float32),
                 jax.ShapeDtypeStruct((N, H2PAD), jnp.float32)],
  )(x, agg1, dis16, w0c1, w1c1, b1, wc21)

  agg2 = _make_lap_sc(H2PAD)(y2, w2, row2, col2)

  out = pl.pallas_call(
      _tc_cell2,
      grid=_GRID,
      in_specs=[_row_spec(H2PAD), _aggp_spec(H2PAD), _row_spec(16),
                _full_spec((H2PAD, 3 * H2)), _full_spec((H2PAD, 3 * H2)),
                _full_spec((1, 3 * H2)), _full_spec((1, H2)),
                _full_spec((H2, 1)), _full_spec((1, 1))],
      out_specs=_row_spec(1),
      out_shape=jax.ShapeDtypeStruct((N, 1), jnp.float32),
  )(h1p, agg2, dis16, w0c2, w1c2, b2, wc22, lin_W, linb)

  return out
